# Initial kernel scaffold; baseline (speedup 1.0000x reference)
import jax
import jax.numpy as jnp
from jax.experimental import pallas as pl


def kernel(inputs, W):
    raise NotImplementedError("write your pallas kernel here")


# speedup vs baseline: 1.1051x; 1.1051x over previous
import functools

import jax
import jax.numpy as jnp
from jax import lax
from jax.experimental import pallas as pl
from jax.experimental.pallas import tpu as pltpu
from jax.experimental.pallas import tpu_sc as plsc

_K = 8192
_D = 64
_N = 8192
_TB = 256
_NT = _N // _TB
_COMMIT = 0.25


def _vq_body(x_ref, idx_ref, wq_ref, enc_ref, qst_ref, loss_ref, perp_ref,
             counts, acc):
    i = pl.program_id(0)
    x = x_ref[0]
    idx = idx_ref[...]
    cols = lax.broadcasted_iota(jnp.int32, (_TB, _K), 1)
    enc = (cols == idx).astype(jnp.float32)
    enc_ref[...] = enc
    q = jax.lax.dot_general(enc, wq_ref[...], (((1,), (0,)), ((), ())),
                            preferred_element_type=jnp.float32)
    qst_ref[...] = (x + (q - x))[None]

    @pl.when(i == 0)
    def _init():
        counts[...] = jnp.zeros_like(counts)
        acc[0] = 0.0

    counts[...] += jnp.sum(enc, axis=0, keepdims=True)
    acc[0] += jnp.sum((q - x) ** 2)

    @pl.when(i == _NT - 1)
    def _finish():
        m = acc[0] / (_N * _D)
        loss_ref[...] = jnp.reshape(m + _COMMIT * m, (1, 1))
        probs = counts[...] * (1.0 / _N)
        ent = jnp.sum(probs * jnp.log(probs + 1e-10))
        perp_ref[...] = jnp.reshape(jnp.exp(-ent), (1, 1))


def kernel(inputs, W):
    x = inputs.reshape(-1, _D)
    distances = (jnp.sum(x ** 2, axis=1, keepdims=True)
                 + jnp.sum(W ** 2, axis=1)
                 - 2.0 * jnp.matmul(x, W.T))
    idx = jnp.argmin(distances, axis=1)
    idx = lax.optimization_barrier(idx)
    idxp = idx.astype(jnp.int32)[:, None]

    nb = 1024 // _TB
    enc, qst, loss, perp = pl.pallas_call(
        _vq_body,
        grid=(_NT,),
        in_specs=[
            pl.BlockSpec((1, _TB, _D), lambda i: (i // nb, i % nb, 0)),
            pl.BlockSpec((_TB, 1), lambda i: (i, 0)),
            pl.BlockSpec((_K, _D), lambda i: (0, 0)),
        ],
        out_specs=(
            pl.BlockSpec((_TB, _K), lambda i: (i, 0)),
            pl.BlockSpec((1, _TB, _D), lambda i: (i // nb, i % nb, 0)),
            pl.BlockSpec((1, 1), lambda i: (0, 0)),
            pl.BlockSpec((1, 1), lambda i: (0, 0)),
        ),
        out_shape=(
            jax.ShapeDtypeStruct((_N, _K), jnp.float32),
            jax.ShapeDtypeStruct(inputs.shape, jnp.float32),
            jax.ShapeDtypeStruct((1, 1), jnp.float32),
            jax.ShapeDtypeStruct((1, 1), jnp.float32),
        ),
        scratch_shapes=[
            pltpu.VMEM((1, _K), jnp.float32),
            pltpu.SMEM((1,), jnp.float32),
        ],
    )(inputs, idxp, W)
    return (loss[0, 0], qst, perp[0, 0], enc)


# speedup vs baseline: 1.1260x; 1.0189x over previous
import functools

import jax
import jax.numpy as jnp
from jax import lax
from jax.experimental import pallas as pl
from jax.experimental.pallas import tpu as pltpu
from jax.experimental.pallas import tpu_sc as plsc

_K = 8192
_D = 64
_N = 8192
_TB = 512
_NT = _N // _TB
_COMMIT = 0.25


def _vq_body(x_ref, idx_ref, wq_ref, enc_ref, qst_ref, loss_ref, perp_ref,
             counts, acc):
    i = pl.program_id(0)
    x = x_ref[0]
    idx = idx_ref[...]
    cols = lax.broadcasted_iota(jnp.int32, (_TB, _K), 1)
    enc = (cols == idx).astype(jnp.float32)
    enc_ref[...] = enc
    q = jax.lax.dot_general(enc, wq_ref[...], (((1,), (0,)), ((), ())),
                            preferred_element_type=jnp.float32)
    qst_ref[...] = (x + (q - x))[None]

    @pl.when(i == 0)
    def _init():
        counts[...] = jnp.zeros_like(counts)
        acc[0] = 0.0

    counts[...] += jnp.sum(enc, axis=0, keepdims=True)
    acc[0] += jnp.sum((q - x) ** 2)

    @pl.when(i == _NT - 1)
    def _finish():
        m = acc[0] / (_N * _D)
        loss_ref[...] = jnp.reshape(m + _COMMIT * m, (1, 1))
        probs = counts[...] * (1.0 / _N)
        ent = jnp.sum(probs * jnp.log(probs + 1e-10))
        perp_ref[...] = jnp.reshape(jnp.exp(-ent), (1, 1))


def kernel(inputs, W):
    x = inputs.reshape(-1, _D)
    distances = (jnp.sum(x ** 2, axis=1, keepdims=True)
                 + jnp.sum(W ** 2, axis=1)
                 - 2.0 * jnp.matmul(x, W.T))
    idx = jnp.argmin(distances, axis=1)
    idx = lax.optimization_barrier(idx)
    idxp = idx.astype(jnp.int32)[:, None]

    nb = 1024 // _TB
    enc, qst, loss, perp = pl.pallas_call(
        _vq_body,
        grid=(_NT,),
        in_specs=[
            pl.BlockSpec((1, _TB, _D), lambda i: (i // nb, i % nb, 0)),
            pl.BlockSpec((_TB, 1), lambda i: (i, 0)),
            pl.BlockSpec((_K, _D), lambda i: (0, 0)),
        ],
        out_specs=(
            pl.BlockSpec((_TB, _K), lambda i: (i, 0)),
            pl.BlockSpec((1, _TB, _D), lambda i: (i // nb, i % nb, 0)),
            pl.BlockSpec((1, 1), lambda i: (0, 0)),
            pl.BlockSpec((1, 1), lambda i: (0, 0)),
        ),
        out_shape=(
            jax.ShapeDtypeStruct((_N, _K), jnp.float32),
            jax.ShapeDtypeStruct(inputs.shape, jnp.float32),
            jax.ShapeDtypeStruct((1, 1), jnp.float32),
            jax.ShapeDtypeStruct((1, 1), jnp.float32),
        ),
        scratch_shapes=[
            pltpu.VMEM((1, _K), jnp.float32),
            pltpu.SMEM((1,), jnp.float32),
        ],
    )(inputs, idxp, W)
    return (loss[0, 0], qst, perp[0, 0], enc)


# speedup vs baseline: 1.1269x; 1.0008x over previous
import jax
import jax.numpy as jnp
from jax import lax
from jax.experimental import pallas as pl
from jax.experimental.pallas import tpu as pltpu

_K = 8192
_D = 64
_N = 8192
_TB = 512
_NT = _N // _TB
_COMMIT = 0.25


def _vq_body(x_ref, idx_ref, wq_ref, enc_ref, qst_ref, loss_ref, perp_ref,
             counts, acc):
    i = pl.program_id(0)
    x = x_ref[0]
    idx = idx_ref[...]
    cols = lax.broadcasted_iota(jnp.int32, (_TB, _K), 1)
    enc = (cols == idx).astype(jnp.float32)
    enc_ref[...] = enc
    q = jax.lax.dot_general(enc, wq_ref[...], (((1,), (0,)), ((), ())),
                            preferred_element_type=jnp.float32)
    qst_ref[...] = (x + (q - x))[None]

    @pl.when(i == 0)
    def _init():
        counts[...] = jnp.zeros_like(counts)
        acc[0] = 0.0

    counts[...] += jnp.sum(enc, axis=0, keepdims=True)
    acc[0] += jnp.sum((q - x) ** 2)

    @pl.when(i == _NT - 1)
    def _finish():
        m = acc[0] / (_N * _D)
        loss_ref[...] = jnp.reshape(m + _COMMIT * m, (1, 1))
        probs = counts[...] * (1.0 / _N)
        ent = jnp.sum(probs * jnp.log(probs + 1e-10))
        perp_ref[...] = jnp.reshape(jnp.exp(-ent), (1, 1))


def kernel(inputs, W):
    x = inputs.reshape(-1, _D)
    distances = (jnp.sum(x ** 2, axis=1, keepdims=True)
                 + jnp.sum(W ** 2, axis=1)
                 - 2.0 * jnp.matmul(x, W.T))
    idx = jnp.argmin(distances, axis=1)
    idx = lax.optimization_barrier(idx)
    idxp = idx.astype(jnp.int32)[:, None]

    nb = 1024 // _TB
    enc, qst, loss, perp = pl.pallas_call(
        _vq_body,
        grid=(_NT,),
        in_specs=[
            pl.BlockSpec((1, _TB, _D), lambda i: (i // nb, i % nb, 0)),
            pl.BlockSpec((_TB, 1), lambda i: (i, 0)),
            pl.BlockSpec((_K, _D), lambda i: (0, 0)),
        ],
        out_specs=(
            pl.BlockSpec((_TB, _K), lambda i: (i, 0)),
            pl.BlockSpec((1, _TB, _D), lambda i: (i // nb, i % nb, 0)),
            pl.BlockSpec((1, 1), lambda i: (0, 0)),
            pl.BlockSpec((1, 1), lambda i: (0, 0)),
        ),
        out_shape=(
            jax.ShapeDtypeStruct((_N, _K), jnp.float32),
            jax.ShapeDtypeStruct(inputs.shape, jnp.float32),
            jax.ShapeDtypeStruct((1, 1), jnp.float32),
            jax.ShapeDtypeStruct((1, 1), jnp.float32),
        ),
        scratch_shapes=[
            pltpu.VMEM((1, _K), jnp.float32),
            pltpu.SMEM((1,), jnp.float32),
        ],
    )(inputs, idxp, W)
    return (loss[0, 0], qst, perp[0, 0], enc)
